# baseline (speedup 1.0000x reference)
import jax
import jax.numpy as jnp
from jax.experimental import pallas as pl
from jax.experimental.pallas import tpu as pltpu

L, R, T, H, D = 4, 16, 2048, 8, 64
A = 4
HD = H * D
LR = L * R


def _row_copy(src_ref, out_ref, cl_ref, ai_ref, c, i, sem):
    li = i // A
    a = jax.lax.rem(i, A)
    r = jax.lax.rem(li, R)
    base = cl_ref[r]
    src = base + ai_ref[r, a]
    return pltpu.make_async_copy(
        src_ref.at[li, pl.ds(src, 1), :, :],
        out_ref.at[c * LR + li, pl.ds(base + a, 1), :, :],
        sem,
    )


def _dma_kernel(cl_ref, ai_ref, k_ref, v_ref, out_ref, bsem, rsem):
    bk = pltpu.make_async_copy(k_ref, out_ref.at[pl.ds(0, LR)], bsem)
    bv = pltpu.make_async_copy(v_ref, out_ref.at[pl.ds(LR, LR)], bsem)
    bk.start()
    bv.start()
    bk.wait()
    bv.wait()

    def start_k(i, _):
        _row_copy(k_ref, out_ref, cl_ref, ai_ref, 0, i, rsem).start()
        return 0

    def start_v(i, _):
        _row_copy(v_ref, out_ref, cl_ref, ai_ref, 1, i, rsem).start()
        return 0

    def wait_k(i, _):
        _row_copy(k_ref, out_ref, cl_ref, ai_ref, 0, i, rsem).wait()
        return 0

    def wait_v(i, _):
        _row_copy(v_ref, out_ref, cl_ref, ai_ref, 1, i, rsem).wait()
        return 0

    jax.lax.fori_loop(0, LR * A, start_k, 0)
    jax.lax.fori_loop(0, LR * A, start_v, 0)
    jax.lax.fori_loop(0, LR * A, wait_k, 0)
    jax.lax.fori_loop(0, LR * A, wait_v, 0)


def kernel(K_cache, V_cache, cachelen, accept_indices):
    Kr = jax.lax.bitcast_convert_type(
        K_cache.reshape(LR, T, 1, HD // 2, 2), jnp.int32)
    Vr = jax.lax.bitcast_convert_type(
        V_cache.reshape(LR, T, 1, HD // 2, 2), jnp.int32)
    out = pl.pallas_call(
        _dma_kernel,
        in_specs=[
            pl.BlockSpec(memory_space=pltpu.SMEM),
            pl.BlockSpec(memory_space=pltpu.SMEM),
            pl.BlockSpec(memory_space=pl.ANY),
            pl.BlockSpec(memory_space=pl.ANY),
        ],
        out_specs=pl.BlockSpec(memory_space=pl.ANY),
        out_shape=jax.ShapeDtypeStruct((2 * LR, T, 1, HD // 2), jnp.int32),
        scratch_shapes=[pltpu.SemaphoreType.DMA, pltpu.SemaphoreType.DMA],
    )(cachelen, accept_indices, Kr, Vr)
    out = jax.lax.bitcast_convert_type(out, K_cache.dtype)
    return out.reshape(2, L, R, T, H, D)


# speedup vs baseline: 3.2104x; 3.2104x over previous
import jax
import jax.numpy as jnp
from jax.experimental import pallas as pl
from jax.experimental.pallas import tpu as pltpu

L, R, T, H, D = 4, 16, 2048, 8, 64
A = 4
HD = H * D
LR = L * R
W = HD // 2


def _fix_window(src_ref, base, r, accept_ref, out_ref):
    aligned = pl.multiple_of((base // 8) * 8, 8)
    off = base - aligned
    win = src_ref[0, pl.ds(aligned, 16), :]
    rows = jax.lax.broadcasted_iota(jnp.int32, (16, 1), 0)
    new = win
    for a in range(A):
        src = off + accept_ref[r, a]
        tgt = off + a
        row = jnp.sum(jnp.where(rows == src, win, 0), axis=0, keepdims=True)
        new = jnp.where(rows == tgt, row, new)
    out_ref[0, pl.ds(aligned, 16), :] = new


def _copy_fix_kernel(cachelen_ref, accept_ref, k_ref, v_ref, out_ref):
    c = pl.program_id(0)
    i = pl.program_id(1)
    r = jax.lax.rem(i, R)
    base = cachelen_ref[r]

    @pl.when(c == 0)
    def _():
        out_ref[...] = k_ref[...]
        _fix_window(k_ref, base, r, accept_ref, out_ref)

    @pl.when(c == 1)
    def _():
        out_ref[...] = v_ref[...]
        _fix_window(v_ref, base, r, accept_ref, out_ref)


def kernel(K_cache, V_cache, cachelen, accept_indices):
    Kr = jax.lax.bitcast_convert_type(
        K_cache.reshape(LR, T, W, 2), jnp.int32)
    Vr = jax.lax.bitcast_convert_type(
        V_cache.reshape(LR, T, W, 2), jnp.int32)
    grid_spec = pltpu.PrefetchScalarGridSpec(
        num_scalar_prefetch=2,
        grid=(2, LR),
        in_specs=[
            pl.BlockSpec((1, T, W), lambda c, i, cl, ai: (i * (1 - c), 0, 0)),
            pl.BlockSpec((1, T, W), lambda c, i, cl, ai: (i * c, 0, 0)),
        ],
        out_specs=pl.BlockSpec((1, T, W), lambda c, i, cl, ai: (c * LR + i, 0, 0)),
    )
    out = pl.pallas_call(
        _copy_fix_kernel,
        grid_spec=grid_spec,
        out_shape=jax.ShapeDtypeStruct((2 * LR, T, W), jnp.int32),
    )(cachelen, accept_indices, Kr, Vr)
    out = jax.lax.bitcast_convert_type(out, K_cache.dtype)
    return out.reshape(2, L, R, T, H, D)


# speedup vs baseline: 11.2584x; 3.5068x over previous
import jax
import jax.numpy as jnp
from jax.experimental import pallas as pl
from jax.experimental.pallas import tpu as pltpu

L, R, T, H, D = 4, 16, 2048, 8, 64
A = 4
HD = H * D
LR = L * R


def _fix_window(src_ref, base, r, accept_ref, out_ref, win_in, win_out, sem):
    aligned = pl.multiple_of((base // 8) * 8, 8)
    off = base - aligned
    ld = pltpu.make_async_copy(
        src_ref.at[:, pl.ds(aligned, 16), :], win_in, sem)
    ld.start()
    ld.wait()
    win = win_in[0, :, :]
    rows = jax.lax.broadcasted_iota(jnp.int32, (16, 1), 0)
    new = win
    for a in range(A):
        src = off + accept_ref[r, a]
        tgt = off + a
        shift = jax.lax.rem(tgt - src + 16, 16)
        rolled = pltpu.roll(win, shift, 0)
        new = jnp.where(rows == tgt, rolled, new)
    win_out[0, :, :] = new
    st = pltpu.make_async_copy(
        win_out, out_ref.at[:, pl.ds(aligned, 16), :], sem)
    st.start()
    st.wait()


def _copy_fix_kernel(cachelen_ref, accept_ref, k_ref, v_ref, out_ref,
                     win_in, win_out, sem, wsem):
    c = pl.program_id(0)
    i = pl.program_id(1)
    r = jax.lax.rem(i, R)
    base = cachelen_ref[r]

    @pl.when(c == 0)
    def _():
        cp = pltpu.make_async_copy(k_ref, out_ref, sem)
        cp.start()
        cp.wait()
        _fix_window(k_ref, base, r, accept_ref, out_ref, win_in, win_out, wsem)

    @pl.when(c == 1)
    def _():
        cp = pltpu.make_async_copy(v_ref, out_ref, sem)
        cp.start()
        cp.wait()
        _fix_window(v_ref, base, r, accept_ref, out_ref, win_in, win_out, wsem)


def kernel(K_cache, V_cache, cachelen, accept_indices):
    Kr = jax.lax.bitcast_convert_type(K_cache, jnp.bfloat16).reshape(LR, T, HD)
    Vr = jax.lax.bitcast_convert_type(V_cache, jnp.bfloat16).reshape(LR, T, HD)
    grid_spec = pltpu.PrefetchScalarGridSpec(
        num_scalar_prefetch=2,
        grid=(2, LR),
        in_specs=[
            pl.BlockSpec((1, T, HD), lambda c, i, cl, ai: (i * (1 - c), 0, 0)),
            pl.BlockSpec((1, T, HD), lambda c, i, cl, ai: (i * c, 0, 0)),
        ],
        out_specs=pl.BlockSpec((1, T, HD), lambda c, i, cl, ai: (c * LR + i, 0, 0)),
        scratch_shapes=[
            pltpu.VMEM((1, 16, HD), jnp.bfloat16),
            pltpu.VMEM((1, 16, HD), jnp.bfloat16),
            pltpu.SemaphoreType.DMA,
            pltpu.SemaphoreType.DMA,
        ],
    )
    out = pl.pallas_call(
        _copy_fix_kernel,
        grid_spec=grid_spec,
        out_shape=jax.ShapeDtypeStruct((2 * LR, T, HD), jnp.bfloat16),
    )(cachelen, accept_indices, Kr, Vr)
    out = jax.lax.bitcast_convert_type(out, K_cache.dtype)
    return out.reshape(2, L, R, T, H, D)
